# baseline (speedup 1.0000x reference)
import functools

import jax
import jax.numpy as jnp
from jax import lax
from jax.experimental import pallas as pl
from jax.experimental.pallas import tpu as pltpu
from jax.experimental.pallas import tpu_sc as plsc

_NC = 2
_NS = 16
_NW = _NC * _NS
_LANES = 16
_CH = 128


def _activation_16(x):
    ax = jnp.abs(x)
    a = jnp.exp(-ax)
    a2 = a * a
    s = 1.0 / (1.0 + a)
    sig = jnp.where(x >= 0.0, s, 1.0 - s)
    tmag = (1.0 - a2) / (1.0 + a2)
    th = jnp.where(x >= 0.0, tmag, -tmag)
    return x * sig + 0.1 * th


def _sc_gather_act(idx, table, n_per_w, d):
    n = idx.shape[0]
    n_ch = n_per_w // _CH
    mesh = plsc.VectorSubcoreMesh(core_axis_name="c", subcore_axis_name="s")

    @functools.partial(
        pl.kernel,
        mesh=mesh,
        out_type=jax.ShapeDtypeStruct((n, d), jnp.float32),
        scratch_types=[
            pltpu.VMEM((n_per_w,), jnp.int32),
            pltpu.VMEM((_CH, d), jnp.float32),
            pltpu.SemaphoreType.DMA,
        ],
        compiler_params=pltpu.CompilerParams(use_tc_tiling_on_sc=False),
    )
    def k(idx_hbm, table_hbm, out_hbm, idx_v, rows_v, sem):
        wid = lax.axis_index("s") * _NC + lax.axis_index("c")
        base = wid * n_per_w
        pltpu.sync_copy(idx_hbm.at[pl.ds(base, n_per_w)], idx_v)

        def chunk_body(ci, _):
            cbase = ci * _CH
            pltpu.async_copy(
                table_hbm.at[idx_v.at[pl.ds(cbase, _CH)]], rows_v, sem
            ).wait()

            def row_body(r, _):
                for blk in range(d // _LANES):
                    x = rows_v[r, pl.ds(blk * _LANES, _LANES)]
                    rows_v[r, pl.ds(blk * _LANES, _LANES)] = _activation_16(x)
                return 0

            lax.fori_loop(0, _CH, row_body, 0, unroll=False)
            pltpu.sync_copy(rows_v, out_hbm.at[pl.ds(base + cbase, _CH)])
            return 0

        lax.fori_loop(0, n_ch, chunk_body, 0, unroll=False)

    return k(idx, table)


def kernel(input_ids, table):
    b, l = input_ids.shape
    v, d = table.shape
    n = b * l
    idx = input_ids.reshape(n).astype(jnp.int32)
    n_per_w = n // _NW
    out = _sc_gather_act(idx, table, n_per_w, d)
    return out.reshape(b, l, d)


# speedup vs baseline: 1.0986x; 1.0986x over previous
import functools

import jax
import jax.numpy as jnp
from jax import lax
from jax.experimental import pallas as pl
from jax.experimental.pallas import tpu as pltpu
from jax.experimental.pallas import tpu_sc as plsc

_NC = 2
_NS = 16
_NW = _NC * _NS
_CH = 128
_TCB = 8000


def _act_relayout(table):
    v, d = table.shape

    def body(t_ref, o_ref):
        x = t_ref[...]
        y = jax.nn.silu(x) + 0.1 * jnp.tanh(x)
        o_ref[:, : d] = y

    return pl.pallas_call(
        body,
        grid=(v // _TCB,),
        in_specs=[pl.BlockSpec((_TCB, d), lambda i: (i, 0))],
        out_specs=pl.BlockSpec((_TCB, 2 * d), lambda i: (i, 0)),
        out_shape=jax.ShapeDtypeStruct((v, 2 * d), jnp.float32),
    )(table)


def _sc_gather(idx, actdup, d):
    n = idx.shape[0]
    n_per_w = n // _NW
    n_ch = n_per_w // _CH
    mesh = plsc.VectorSubcoreMesh(core_axis_name="c", subcore_axis_name="s")

    @functools.partial(
        pl.kernel,
        mesh=mesh,
        out_type=jax.ShapeDtypeStruct((n, d), jnp.float32),
        scratch_types=[
            pltpu.VMEM((n_per_w,), jnp.int32),
            pltpu.VMEM((_CH, 2 * d), jnp.float32),
            pltpu.VMEM((_CH, 2 * d), jnp.float32),
            pltpu.SemaphoreType.DMA,
            pltpu.SemaphoreType.DMA,
            pltpu.SemaphoreType.DMA,
            pltpu.SemaphoreType.DMA,
        ],
        compiler_params=pltpu.CompilerParams(use_tc_tiling_on_sc=False),
    )
    def k(idx_hbm, src_hbm, out_hbm, idx_v, row0, row1, sg0, sg1, so0, so1):
        wid = lax.axis_index("s") * _NC + lax.axis_index("c")
        base = wid * n_per_w
        pltpu.sync_copy(idx_hbm.at[pl.ds(base, n_per_w)], idx_v)

        rows = (row0, row1)
        sg = (sg0, sg1)
        so = (so0, so1)

        for b in range(2):
            pltpu.async_copy(
                src_hbm.at[idx_v.at[pl.ds(b * _CH, _CH)]], rows[b], sg[b]
            )

        def group(g, _):
            for b in range(2):
                ci = 2 * g + b
                pltpu.make_async_copy(
                    src_hbm.at[idx_v.at[pl.ds(0, _CH)]], rows[b], sg[b]
                ).wait()
                cp = pltpu.async_copy(
                    rows[b].at[:, pl.ds(0, d)],
                    out_hbm.at[pl.ds(base + ci * _CH, _CH)],
                    so[b],
                )
                @pl.when(ci + 2 < n_ch)
                def _():
                    pltpu.make_async_copy(
                        rows[b].at[:, pl.ds(0, d)],
                        out_hbm.at[pl.ds(base, _CH)],
                        so[b],
                    ).wait()
                    pltpu.async_copy(
                        src_hbm.at[idx_v.at[pl.ds((ci + 2) * _CH, _CH)]],
                        rows[b],
                        sg[b],
                    )
            return 0

        lax.fori_loop(0, n_ch // 2, group, 0, unroll=False)
        for b in range(2):
            pltpu.make_async_copy(
                rows[b].at[:, pl.ds(0, d)],
                out_hbm.at[pl.ds(base, _CH)],
                so[b],
            ).wait()

    return k(idx, actdup)


def kernel(input_ids, table):
    b, l = input_ids.shape
    v, d = table.shape
    idx = input_ids.reshape(b * l).astype(jnp.int32)
    actdup = _act_relayout(table)
    out = _sc_gather(idx, actdup, d)
    return out.reshape(b, l, d)
